# Initial kernel scaffold; baseline (speedup 1.0000x reference)
import jax
import jax.numpy as jnp
from jax.experimental import pallas as pl


def kernel(x, table):
    raise NotImplementedError("write your pallas kernel here")


# speedup vs baseline: 1.8451x; 1.8451x over previous
import functools

import jax
import jax.numpy as jnp
from jax import lax
from jax.experimental import pallas as pl
from jax.experimental.pallas import tpu as pltpu
from jax.experimental.pallas import tpu_sc as plsc


def _sc_gather(table, idx_flat):
    info = plsc.get_sparse_core_info()
    nw = info.num_cores * info.num_subcores
    n = idx_flat.shape[0]
    d = table.shape[1]
    b_per_w = n // nw
    chunk = 1024
    n_chunks = b_per_w // chunk
    nc = info.num_cores

    mesh = plsc.VectorSubcoreMesh(core_axis_name="c", subcore_axis_name="s")

    @functools.partial(
        pl.kernel,
        mesh=mesh,
        compiler_params=pltpu.CompilerParams(use_tc_tiling_on_sc=False),
        out_type=jax.ShapeDtypeStruct((n, d), jnp.float32),
        scratch_types=[
            pltpu.VMEM((chunk,), jnp.int32),
            pltpu.VMEM((chunk, d), jnp.float32),
            pltpu.SemaphoreType.DMA,
        ],
    )
    def k(table_hbm, idx_hbm, out_hbm, idx_v, rows_v, sem):
        wid = lax.axis_index("s") * nc + lax.axis_index("c")
        base = wid * b_per_w

        def body(i, carry):
            off = base + i * chunk
            pltpu.sync_copy(idx_hbm.at[pl.ds(off, chunk)], idx_v)
            pltpu.async_copy(table_hbm.at[idx_v], rows_v, sem).wait()
            pltpu.sync_copy(rows_v, out_hbm.at[pl.ds(off, chunk)])
            return carry

        lax.fori_loop(0, n_chunks, body, 0)

    return k(table, idx_flat)


def kernel(x, table):
    idx_flat = x.reshape(-1).astype(jnp.int32)
    out = _sc_gather(table, idx_flat)
    return out.reshape(x.shape[0], x.shape[1], table.shape[1])


# speedup vs baseline: 1.9941x; 1.0808x over previous
import functools

import jax
import jax.numpy as jnp
from jax import lax
from jax.experimental import pallas as pl
from jax.experimental.pallas import tpu as pltpu
from jax.experimental.pallas import tpu_sc as plsc

_V = 1000000
_D = 64
_B = 16384
_L = 50


def _transpose_table(table):
    tt = table.T
    cb = 8192
    grid = -(-_V // cb)

    def body(in_ref, out_ref):
        x = in_ref[...]
        t3 = x.T.reshape(cb // 2, 2, _D)
        out_ref[...] = jnp.concatenate([t3[:, 0, :], t3[:, 1, :]], axis=1)

    return pl.pallas_call(
        body,
        grid=(grid,),
        in_specs=[pl.BlockSpec((_D, cb), lambda j: (0, j))],
        out_specs=pl.BlockSpec((cb // 2, 128), lambda j: (j, 0)),
        out_shape=jax.ShapeDtypeStruct((_V // 2, 128), jnp.float32),
    )(tt)


def _sc_gather(table_lin, idx_flat):
    info = plsc.get_sparse_core_info()
    nc = info.num_cores
    nw = nc * info.num_subcores
    n = idx_flat.shape[0]
    b_per_w = n // nw
    chunk = 1024
    n_chunks = b_per_w // chunk

    mesh = plsc.VectorSubcoreMesh(core_axis_name="c", subcore_axis_name="s")

    @functools.partial(
        pl.kernel,
        mesh=mesh,
        compiler_params=pltpu.CompilerParams(use_tc_tiling_on_sc=False),
        out_type=jax.ShapeDtypeStruct((n, _D), jnp.float32),
        scratch_types=[
            pltpu.VMEM((chunk,), jnp.int32),
            pltpu.VMEM((chunk, _D), jnp.float32),
            pltpu.SemaphoreType.DMA,
        ],
    )
    def k(table_hbm, idx_hbm, out_hbm, idx_v, rows_v, sem):
        wid = lax.axis_index("s") * nc + lax.axis_index("c")
        base = wid * b_per_w

        def body(i, carry):
            off = base + i * chunk
            pltpu.sync_copy(idx_hbm.at[pl.ds(off, chunk)], idx_v)
            pltpu.async_copy(table_hbm.at[idx_v], rows_v, sem).wait()
            pltpu.sync_copy(rows_v, out_hbm.at[pl.ds(off, chunk)])
            return carry

        lax.fori_loop(0, n_chunks, body, 0)

    return k(table_lin, idx_flat)


def _relayout_out(rows2d):
    bl = 512
    nb = _B // bl
    rpb = bl * _L * _D // 128

    def body(in_ref, out_ref):
        v = in_ref[...].reshape(bl, _L * _D // 128, 128)
        out_ref[...] = jnp.transpose(v, (1, 2, 0)).reshape(_L, _D, bl)

    return pl.pallas_call(
        body,
        grid=(nb,),
        in_specs=[pl.BlockSpec((rpb, 128), lambda j: (j, 0))],
        out_specs=pl.BlockSpec((_L, _D, bl), lambda j: (0, 0, j)),
        out_shape=jax.ShapeDtypeStruct((_L, _D, _B), jnp.float32),
    )(rows2d)


def kernel(x, table):
    table_lin = _transpose_table(table).reshape(_V, _D)
    idx_flat = x.reshape(-1).astype(jnp.int32)
    rows = _sc_gather(table_lin, idx_flat)
    out = _relayout_out(rows.reshape(_B * _L * _D // 128, 128))
    return jnp.transpose(out, (2, 0, 1))


# speedup vs baseline: 3.2176x; 1.6135x over previous
import functools

import jax
import jax.numpy as jnp
from jax import lax
from jax.experimental import pallas as pl
from jax.experimental.pallas import tpu as pltpu
from jax.experimental.pallas import tpu_sc as plsc

_V = 1000000
_D = 64
_B = 16384
_L = 50
_CB = 8192
_GRID_T = -(-_V // _CB)
_TAIL = (_GRID_T - 1) * _CB


def _transpose_table(table):
    tt = table.T

    def body(in_ref, out_ref):
        j = pl.program_id(0)
        x = in_ref[...]

        @pl.when(j < _GRID_T - 1)
        def _main():
            y = jnp.concatenate([x[:, : _CB // 2], x[:, _CB // 2:]], axis=0)
            out_ref[...] = y.T

        @pl.when(j == _GRID_T - 1)
        def _tail():
            t3 = x.T.reshape(_CB // 2, 2, _D)
            out_ref[...] = jnp.concatenate([t3[:, 0, :], t3[:, 1, :]], axis=1)

    return pl.pallas_call(
        body,
        grid=(_GRID_T,),
        in_specs=[pl.BlockSpec((_D, _CB), lambda j: (0, j))],
        out_specs=pl.BlockSpec((_CB // 2, 128), lambda j: (j, 0)),
        out_shape=jax.ShapeDtypeStruct((_V // 2, 128), jnp.float32),
    )(tt)


def _sc_gather(table_lin, idx_flat):
    info = plsc.get_sparse_core_info()
    nc = info.num_cores
    nw = nc * info.num_subcores
    n = idx_flat.shape[0]
    b_per_w = n // nw
    chunk = 1024
    n_chunks = b_per_w // chunk

    mesh = plsc.VectorSubcoreMesh(core_axis_name="c", subcore_axis_name="s")

    @functools.partial(
        pl.kernel,
        mesh=mesh,
        compiler_params=pltpu.CompilerParams(use_tc_tiling_on_sc=False),
        out_type=jax.ShapeDtypeStruct((n, _D), jnp.float32),
        scratch_types=[
            pltpu.VMEM((chunk,), jnp.int32),
            pltpu.VMEM((chunk,), jnp.int32),
            pltpu.VMEM((chunk, _D), jnp.float32),
            pltpu.SemaphoreType.DMA,
        ],
    )
    def k(table_hbm, idx_hbm, out_hbm, raw_v, idx_v, rows_v, sem):
        wid = lax.axis_index("s") * nc + lax.axis_index("c")
        base = wid * b_per_w

        def body(i, carry):
            off = base + i * chunk
            pltpu.sync_copy(idx_hbm.at[pl.ds(off, chunk)], raw_v)

            def remap(t, c):
                v = raw_v[pl.ds(t * 16, 16)]
                main = (
                    (v & jnp.int32(-_CB))
                    | ((v & jnp.int32(_CB // 2 - 1)) << 1)
                    | (lax.shift_right_logical(v, 12) & jnp.int32(1))
                )
                idx_v[pl.ds(t * 16, 16)] = jnp.where(v >= _TAIL, v, main)
                return c

            lax.fori_loop(0, chunk // 16, remap, 0)
            pltpu.async_copy(table_hbm.at[idx_v], rows_v, sem).wait()
            pltpu.sync_copy(rows_v, out_hbm.at[pl.ds(off, chunk)])
            return carry

        lax.fori_loop(0, n_chunks, body, 0)

    return k(table_lin, idx_flat)


def _relayout_out(rows2d):
    bb = 512
    ld = _L * _D

    def body(in_ref, out_ref):
        out_ref[...] = in_ref[...].T

    return pl.pallas_call(
        body,
        grid=(_B // bb,),
        in_specs=[pl.BlockSpec((bb, ld), lambda j: (j, 0))],
        out_specs=pl.BlockSpec((ld, bb), lambda j: (0, j)),
        out_shape=jax.ShapeDtypeStruct((ld, _B), jnp.float32),
    )(rows2d)


def kernel(x, table):
    table_lin = _transpose_table(table).reshape(_V, _D)
    idx_flat = x.reshape(-1).astype(jnp.int32)
    rows = _sc_gather(table_lin, idx_flat)
    out = _relayout_out(rows.reshape(_B, _L * _D))
    return jnp.transpose(out.reshape(_L, _D, _B), (2, 0, 1))


# speedup vs baseline: 3.8892x; 1.2087x over previous
import functools

import jax
import jax.numpy as jnp
from jax import lax
from jax.experimental import pallas as pl
from jax.experimental.pallas import tpu as pltpu
from jax.experimental.pallas import tpu_sc as plsc

_V = 1000000
_D = 64
_B = 16384
_L = 50
_CB = 8192
_GRID_T = -(-_V // _CB)
_TAIL = (_GRID_T - 1) * _CB


def _transpose_table(table):
    tt = table.T

    def body(in_ref, out_ref):
        j = pl.program_id(0)
        x = in_ref[...]

        @pl.when(j < _GRID_T - 1)
        def _main():
            y = jnp.concatenate([x[:, : _CB // 2], x[:, _CB // 2:]], axis=0)
            out_ref[...] = y.T

        @pl.when(j == _GRID_T - 1)
        def _tail():
            t3 = x.T.reshape(_CB // 2, 2, _D)
            out_ref[...] = jnp.concatenate([t3[:, 0, :], t3[:, 1, :]], axis=1)

    return pl.pallas_call(
        body,
        grid=(_GRID_T,),
        in_specs=[pl.BlockSpec((_D, _CB), lambda j: (0, j))],
        out_specs=pl.BlockSpec((_CB // 2, 128), lambda j: (j, 0)),
        out_shape=jax.ShapeDtypeStruct((_V // 2, 128), jnp.float32),
    )(tt)


def _sc_gather(table_lin, idx_lmajor):
    info = plsc.get_sparse_core_info()
    nc = info.num_cores
    nw = nc * info.num_subcores
    chunk = 1024
    njb = _B // chunk
    units_per_w = _L * njb // nw

    mesh = plsc.VectorSubcoreMesh(core_axis_name="c", subcore_axis_name="s")

    @functools.partial(
        pl.kernel,
        mesh=mesh,
        compiler_params=pltpu.CompilerParams(use_tc_tiling_on_sc=False),
        out_type=jax.ShapeDtypeStruct((_L // 2, _B, 128), jnp.float32),
        scratch_types=[
            pltpu.VMEM((chunk,), jnp.int32),
            pltpu.VMEM((chunk,), jnp.int32),
            pltpu.VMEM((chunk, _D), jnp.float32),
            pltpu.SemaphoreType.DMA,
        ],
    )
    def k(table_hbm, idx_hbm, out_hbm, raw_v, idx_v, rows_v, sem):
        wid = lax.axis_index("s") * nc + lax.axis_index("c")

        def body(i, carry):
            u = wid * units_per_w + i
            l = u // njb
            jb = u % njb
            pltpu.sync_copy(idx_hbm.at[pl.ds(l * _B + jb * chunk, chunk)], raw_v)

            def remap(t, c):
                v = raw_v[pl.ds(t * 16, 16)]
                main = (
                    (v & jnp.int32(-_CB))
                    | ((v & jnp.int32(_CB // 2 - 1)) << 1)
                    | (lax.shift_right_logical(v, 12) & jnp.int32(1))
                )
                idx_v[pl.ds(t * 16, 16)] = jnp.where(v >= _TAIL, v, main)
                return c

            lax.fori_loop(0, chunk // 16, remap, 0)
            pltpu.async_copy(table_hbm.at[idx_v], rows_v, sem).wait()
            pltpu.sync_copy(
                rows_v,
                out_hbm.at[l // 2, pl.ds(jb * chunk, chunk),
                           pl.ds((l % 2) * _D, _D)],
            )
            return carry

        lax.fori_loop(0, units_per_w, body, 0)

    return k(table_lin, idx_lmajor)


def _relayout_out(out3):
    bb = 2048

    def body(in_ref, out_ref):
        out_ref[...] = in_ref[0].T

    return pl.pallas_call(
        body,
        grid=(_L // 2, _B // bb),
        in_specs=[pl.BlockSpec((1, bb, 128), lambda q, j: (q, j, 0))],
        out_specs=pl.BlockSpec((128, bb), lambda q, j: (q, j)),
        out_shape=jax.ShapeDtypeStruct((_L * _D, _B), jnp.float32),
    )(out3)


def kernel(x, table):
    table_lin = _transpose_table(table).reshape(_V, _D)
    idx_lmajor = jnp.transpose(x).reshape(-1).astype(jnp.int32)
    out3 = _sc_gather(table_lin, idx_lmajor)
    out = _relayout_out(out3)
    return jnp.transpose(out.reshape(_L, _D, _B), (2, 0, 1))


# speedup vs baseline: 4.5192x; 1.1620x over previous
import functools

import jax
import jax.numpy as jnp
from jax import lax
from jax.experimental import pallas as pl
from jax.experimental.pallas import tpu as pltpu
from jax.experimental.pallas import tpu_sc as plsc

_V = 1000000
_D = 64
_B = 16384
_L = 50
_CB = 8192
_GRID_T = -(-_V // _CB)
_TAIL = (_GRID_T - 1) * _CB


def _transpose_table(table):
    tt = table.T

    def body(in_ref, out_ref):
        j = pl.program_id(0)
        x = in_ref[...]

        @pl.when(j < _GRID_T - 1)
        def _main():
            y = jnp.concatenate([x[:, : _CB // 2], x[:, _CB // 2:]], axis=0)
            out_ref[...] = y.T

        @pl.when(j == _GRID_T - 1)
        def _tail():
            t3 = x.T.reshape(_CB // 2, 2, _D)
            out_ref[...] = jnp.concatenate([t3[:, 0, :], t3[:, 1, :]], axis=1)

    return pl.pallas_call(
        body,
        grid=(_GRID_T,),
        in_specs=[pl.BlockSpec((_D, _CB), lambda j: (0, j))],
        out_specs=pl.BlockSpec((_CB // 2, 128), lambda j: (j, 0)),
        out_shape=jax.ShapeDtypeStruct((_V // 2, 128), jnp.float32),
    )(tt)


def _sc_gather(table_lin, idx_lmajor):
    info = plsc.get_sparse_core_info()
    nc = info.num_cores
    nw = nc * info.num_subcores
    chunk = 1024
    njb = _B // chunk
    units_per_w = _L * njb // nw

    mesh = plsc.VectorSubcoreMesh(core_axis_name="c", subcore_axis_name="s")

    @functools.partial(
        pl.kernel,
        mesh=mesh,
        compiler_params=pltpu.CompilerParams(use_tc_tiling_on_sc=False),
        out_type=jax.ShapeDtypeStruct((_L // 2, _B, 128), jnp.float32),
        scratch_types=[
            pltpu.VMEM((chunk,), jnp.int32),
            pltpu.VMEM((chunk,), jnp.int32),
            pltpu.VMEM((chunk, _D), jnp.float32),
            pltpu.SemaphoreType.DMA,
        ],
    )
    def k(table_hbm, idx_hbm, out_hbm, raw_v, idx_v, rows_v, sem):
        wid = lax.axis_index("s") * nc + lax.axis_index("c")

        def body(i, carry):
            u = wid * units_per_w + i
            l = u // njb
            jb = u % njb
            pltpu.sync_copy(idx_hbm.at[pl.ds(l * _B + jb * chunk, chunk)], raw_v)

            def remap(t, c):
                v = raw_v[pl.ds(t * 16, 16)]
                main = (
                    (v & jnp.int32(-_CB))
                    | ((v & jnp.int32(_CB // 2 - 1)) << 1)
                    | (lax.shift_right_logical(v, 12) & jnp.int32(1))
                )
                idx_v[pl.ds(t * 16, 16)] = jnp.where(v >= _TAIL, v, main)
                return c

            lax.fori_loop(0, chunk // 16, remap, 0)
            pltpu.async_copy(table_hbm.at[idx_v], rows_v, sem).wait()
            pltpu.sync_copy(
                rows_v,
                out_hbm.at[l // 2, pl.ds(jb * chunk, chunk),
                           pl.ds((l % 2) * _D, _D)],
            )
            return carry

        lax.fori_loop(0, units_per_w, body, 0)

    return k(table_lin, idx_lmajor)


def _relayout_out(out3):
    bb = 8192

    def body(in_ref, out_ref):
        out_ref[...] = in_ref[0].T

    return pl.pallas_call(
        body,
        grid=(_L // 2, _B // bb),
        in_specs=[pl.BlockSpec((1, bb, 128), lambda q, j: (q, j, 0))],
        out_specs=pl.BlockSpec((128, bb), lambda q, j: (q, j)),
        out_shape=jax.ShapeDtypeStruct((_L * _D, _B), jnp.float32),
    )(out3)


def kernel(x, table):
    table_lin = _transpose_table(table).reshape(_V, _D)
    idx_lmajor = jnp.transpose(x).reshape(-1).astype(jnp.int32)
    out3 = _sc_gather(table_lin, idx_lmajor)
    out = _relayout_out(out3)
    return jnp.transpose(out.reshape(_L, _D, _B), (2, 0, 1))
